# baseline (speedup 1.0000x reference)
import functools
import math

import jax
import jax.numpy as jnp
from jax import lax
from jax.experimental import pallas as pl
from jax.experimental.pallas import tpu as pltpu
from jax.experimental.pallas import tpu_sc as plsc

_ATOM_K = 15
_EDGE_DIM = 8
_HID = 128


_BIG = 2 ** 30


def _knn_idx(pos_r, posT, bc, br, bs, idx_ref, dr_ref, ir_ref, scr, *,
             rb, n, cb):
    i = pl.program_id(0)
    nchunks = n // cb
    inf = jnp.inf
    fake = 1e30
    pr = pos_r[...]
    bcv = bc[...]
    rid = jax.lax.broadcasted_iota(jnp.int32, (rb, 1), 0) + i * rb
    dr_ref[...] = jnp.full((rb, 16), inf, jnp.float32)
    ir_ref[...] = jnp.full((rb, 16), fake, jnp.float32)
    bmin_r = bs[i * rb]
    bmax_r = bs[i * rb + rb - 1]
    bvals = jax.lax.broadcasted_iota(jnp.int32, (16, 1), 0)
    cnt = jnp.sum((bvals == br[...]).astype(jnp.float32), axis=1,
                  keepdims=True)
    need0 = jnp.max(((bvals >= bmin_r) & (bvals <= bmax_r)
                     & (cnt < 16.0)).astype(jnp.int32)) > 0

    def chunk_body(c, _):
        lo = pl.multiple_of(c * cb, cb)
        overlap = (bs[lo] <= bmax_r) & (bs[lo + cb - 1] >= bmin_r)

        @pl.when(overlap | ((c == 0) & need0))
        def _():
            acc = None
            for d in range(3):
                diff = pr[:, d:d + 1] - posT[d:d + 1, pl.ds(lo, cb)]
                sq = diff * diff
                acc = sq if acc is None else acc + sq
            cidc = jax.lax.broadcasted_iota(jnp.int32, (1, cb), 1) + lo
            cidf = cidc.astype(jnp.float32)
            valid = (bcv == br[:, pl.ds(lo, cb)]) & (rid != cidc)
            scr[...] = jnp.where(valid, acc, inf)
            cd_cols, ci_cols = [], []
            for j in range(_ATOM_K):
                v = scr[...]
                m = jnp.min(v, axis=1, keepdims=True)
                cand = jnp.where(v == m, cidf, fake)
                am = jnp.min(cand, axis=1, keepdims=True)
                cd_cols.append(m)
                ci_cols.append(am)
                if j + 1 < _ATOM_K:
                    scr[...] = jnp.where(cidf == am, inf, v)
            pad_d = jnp.full((rb, 1), inf, jnp.float32)
            pad_i = jnp.full((rb, 1), fake, jnp.float32)
            d_cat = jnp.concatenate([dr_ref[...]] + cd_cols + [pad_d], axis=1)
            i_cat = jnp.concatenate([ir_ref[...]] + ci_cols + [pad_i], axis=1)
            nd, ni = [], []
            for j in range(_ATOM_K):
                m2 = jnp.min(d_cat, axis=1, keepdims=True)
                c2 = jnp.where(d_cat == m2, i_cat, fake)
                am2 = jnp.min(c2, axis=1, keepdims=True)
                nd.append(m2)
                ni.append(am2)
                if j + 1 < _ATOM_K:
                    d_cat = jnp.where((d_cat == m2) & (i_cat == am2), inf,
                                      d_cat)
            dr_ref[...] = jnp.concatenate(nd + [pad_d], axis=1)
            ir_ref[...] = jnp.concatenate(ni + [pad_i], axis=1)
        return ()

    lax.fori_loop(0, nchunks, chunk_body, (), unroll=False)
    iv = ir_ref[...]
    zero = jnp.zeros((rb, 1), jnp.float32)
    idx_ref[...] = jnp.concatenate(
        [iv[:, :_ATOM_K], zero], axis=1).astype(jnp.int32)


def _knn(pos, batch32, rb=1024, cb=1024):
    n = pos.shape[0]
    pos_pad = jnp.pad(pos, ((0, 0), (0, 5)))
    posT = jnp.pad(pos.T, ((0, 5), (0, 0)))
    bc = batch32[:, None]
    br = batch32[None, :]
    nchunks = n // cb
    grid = (n // rb,)
    out = pl.pallas_call(
        functools.partial(_knn_idx, rb=rb, n=n, cb=cb),
        grid=grid,
        in_specs=[
            pl.BlockSpec((rb, 8), lambda i: (i, 0)),
            pl.BlockSpec((8, n), lambda i: (0, 0)),
            pl.BlockSpec((rb, 1), lambda i: (i, 0)),
            pl.BlockSpec((1, n), lambda i: (0, 0)),
            pl.BlockSpec(memory_space=pltpu.SMEM),
        ],
        out_specs=pl.BlockSpec((rb, 16), lambda i: (i, 0)),
        out_shape=jax.ShapeDtypeStruct((n, 16), jnp.int32),
        scratch_shapes=[
            pltpu.VMEM((rb, 16), jnp.float32),
            pltpu.VMEM((rb, 16), jnp.float32),
            pltpu.VMEM((rb, cb), jnp.float32),
        ],
    )(pos_pad, posT, bc, br, batch32)
    return out[:, :_ATOM_K]


def _sc_gather(table, idx_flat):
    b = idx_flat.shape[0]
    d = table.shape[1]
    info = plsc.get_sparse_core_info()
    nw = info.num_cores * info.num_subcores
    nc = info.num_cores
    b_per_w = b // nw
    chunk = 640
    nchunks = b_per_w // chunk
    mesh = plsc.VectorSubcoreMesh(core_axis_name="c", subcore_axis_name="s")

    @functools.partial(
        pl.kernel, mesh=mesh,
        out_type=jax.ShapeDtypeStruct((b, d), jnp.float32),
        scratch_types=[
            pltpu.VMEM((chunk,), jnp.int32),
            pltpu.VMEM((chunk, d), jnp.float32),
            pltpu.SemaphoreType.DMA,
        ],
    )
    def gather_k(table_hbm, idx_hbm, out_hbm, idx_v, rows_v, sem):
        wid = lax.axis_index("s") * nc + lax.axis_index("c")
        base = wid * b_per_w
        for c in range(nchunks):
            off = base + c * chunk
            pltpu.sync_copy(idx_hbm.at[pl.ds(off, chunk)], idx_v)
            pltpu.async_copy(table_hbm.at[idx_v], rows_v, sem).wait()
            pltpu.sync_copy(rows_v, out_hbm.at[pl.ds(off, chunk)])

    return gather_k(table, idx_flat)


def _attn_body(x_r, pos_r, xg_r, cen, W1e, b1e, W2e, b2e, Wq, Wk, We, Wv, Wo,
               out_ref, vscr, *, rb):
    f32 = jnp.float32
    q = jnp.dot(x_r[...], Wq[...], preferred_element_type=f32)
    pr = pos_r[...]
    svec = []
    for j in range(_ATOM_K):
        g = xg_r[j]
        xn = g[:, :64]
        acc = None
        for dd in range(3):
            diff = g[:, 64 + dd:65 + dd] - pr[:, dd:dd + 1]
            sq = diff * diff
            acc = sq if acc is None else acc + sq
        dist = jnp.sqrt(acc + 1e-8)
        delta = dist - cen[...]
        rbf = jnp.exp(-2.0 * delta * delta)
        h1 = jnp.maximum(
            jnp.dot(rbf, W1e[...], preferred_element_type=f32) + b1e[...], 0.0)
        h = jnp.dot(h1, W2e[...], preferred_element_type=f32) + b2e[...]
        km = (jnp.dot(xn, Wk[...], preferred_element_type=f32)
              + jnp.dot(h, We[...], preferred_element_type=f32))
        vj = jnp.dot(xn, Wv[...], preferred_element_type=f32)
        vscr[j] = vj
        s = jnp.sum(q * km, axis=1, keepdims=True) / math.sqrt(float(_HID))
        svec.append(s)
    sc_all = jnp.concatenate(svec, axis=1)
    m = jnp.max(sc_all, axis=1, keepdims=True)
    e = jnp.exp(sc_all - m)
    alpha = e / jnp.sum(e, axis=1, keepdims=True)
    agg = None
    for j in range(_ATOM_K):
        t = alpha[:, j:j + 1] * vscr[j]
        agg = t if agg is None else agg + t
    out_ref[...] = jnp.dot(agg, Wo[...], preferred_element_type=f32)


def _attn(x, pos, xg3, cen, W1e, b1e, W2e, b2e, Wq, Wk, We, Wv, Wo, rb=256):
    n, feat = x.shape
    hid = Wq.shape[1]
    out_dim = Wo.shape[1]
    pos_pad = jnp.pad(pos, ((0, 0), (0, 5)))
    grid = (n // rb,)
    full = lambda shape: pl.BlockSpec(shape, lambda i: tuple(0 for _ in shape))
    return pl.pallas_call(
        functools.partial(_attn_body, rb=rb),
        grid=grid,
        in_specs=[
            pl.BlockSpec((rb, feat), lambda i: (i, 0)),
            pl.BlockSpec((rb, 8), lambda i: (i, 0)),
            pl.BlockSpec((_ATOM_K, rb, 128), lambda i: (0, i, 0)),
            full((1, 8)),
            full((8, 8)), full((1, 8)), full((8, 8)), full((1, 8)),
            full((feat, hid)), full((feat, hid)), full((8, hid)),
            full((feat, hid)), full((hid, out_dim)),
        ],
        out_specs=pl.BlockSpec((rb, out_dim), lambda i: (i, 0)),
        out_shape=jax.ShapeDtypeStruct((n, out_dim), jnp.float32),
        scratch_shapes=[pltpu.VMEM((_ATOM_K, rb, hid), jnp.float32)],
    )(x, pos_pad, xg3, cen, W1e, b1e, W2e, b2e, Wq, Wk, We, Wv, Wo)


def kernel(x, pos, batch, W1e, b1e, W2e, b2e, Wq, Wk, We, Wv, Wo):
    n, feat = x.shape
    batch32 = batch.astype(jnp.int32)
    idx = _knn(pos, batch32)
    idx_flat = idx.T.reshape(-1).astype(jnp.int32)
    table = jnp.concatenate(
        [x, pos, jnp.zeros((n, 128 - feat - 3), jnp.float32)], axis=1)
    xg = _sc_gather(table, idx_flat)
    xg3 = xg.reshape(_ATOM_K, n, 128)
    cen = jnp.linspace(0.0, 6.0, _EDGE_DIM, dtype=jnp.float32)[None, :]
    return _attn(x, pos, xg3, cen, W1e, b1e[None, :], W2e, b2e[None, :],
                 Wq, Wk, We, Wv, Wo)


# speedup vs baseline: 1.2410x; 1.2410x over previous
import functools
import math

import jax
import jax.numpy as jnp
from jax import lax
from jax.experimental import pallas as pl
from jax.experimental.pallas import tpu as pltpu
from jax.experimental.pallas import tpu_sc as plsc

_ATOM_K = 15
_EDGE_DIM = 8
_HID = 128


_BIG = 2 ** 30


def _knn_idx(pos_r, posT, bc, br, bs, idx_ref, dr_ref, ir_ref, scr, *,
             rb, n, cb):
    i = pl.program_id(0)
    nchunks = n // cb
    inf = jnp.inf
    fake = 1e30
    pr = pos_r[...]
    bcv = bc[...]
    rid = jax.lax.broadcasted_iota(jnp.int32, (rb, 1), 0) + i * rb
    dr_ref[...] = jnp.full((rb, 16), inf, jnp.float32)
    ir_ref[...] = jnp.full((rb, 16), fake, jnp.float32)
    bmin_r = bs[i * rb]
    bmax_r = bs[i * rb + rb - 1]
    bvals = jax.lax.broadcasted_iota(jnp.int32, (16, 1), 0)
    cnt = jnp.sum((bvals == br[...]).astype(jnp.float32), axis=1,
                  keepdims=True)
    need0 = jnp.max(((bvals >= bmin_r) & (bvals <= bmax_r)
                     & (cnt < 16.0)).astype(jnp.int32)) > 0

    def chunk_body(c, _):
        lo = pl.multiple_of(c * cb, cb)
        overlap = (bs[lo] <= bmax_r) & (bs[lo + cb - 1] >= bmin_r)

        @pl.when(overlap | ((c == 0) & need0))
        def _():
            acc = None
            for d in range(3):
                diff = pr[:, d:d + 1] - posT[d:d + 1, pl.ds(lo, cb)]
                sq = diff * diff
                acc = sq if acc is None else acc + sq
            cidc = jax.lax.broadcasted_iota(jnp.int32, (1, cb), 1) + lo
            cidf = cidc.astype(jnp.float32)
            valid = (bcv == br[:, pl.ds(lo, cb)]) & (rid != cidc)
            scr[...] = jnp.where(valid, acc, inf)
            cd_cols, ci_cols = [], []
            for j in range(_ATOM_K):
                v = scr[...]
                m = jnp.min(v, axis=1, keepdims=True)
                cand = jnp.where(v == m, cidf, fake)
                am = jnp.min(cand, axis=1, keepdims=True)
                cd_cols.append(m)
                ci_cols.append(am)
                if j + 1 < _ATOM_K:
                    scr[...] = jnp.where(cidf == am, inf, v)
            pad_d = jnp.full((rb, 1), inf, jnp.float32)
            pad_i = jnp.full((rb, 1), fake, jnp.float32)
            d_cat = jnp.concatenate([dr_ref[...]] + cd_cols + [pad_d], axis=1)
            i_cat = jnp.concatenate([ir_ref[...]] + ci_cols + [pad_i], axis=1)
            nd, ni = [], []
            for j in range(_ATOM_K):
                m2 = jnp.min(d_cat, axis=1, keepdims=True)
                c2 = jnp.where(d_cat == m2, i_cat, fake)
                am2 = jnp.min(c2, axis=1, keepdims=True)
                nd.append(m2)
                ni.append(am2)
                if j + 1 < _ATOM_K:
                    d_cat = jnp.where((d_cat == m2) & (i_cat == am2), inf,
                                      d_cat)
            dr_ref[...] = jnp.concatenate(nd + [pad_d], axis=1)
            ir_ref[...] = jnp.concatenate(ni + [pad_i], axis=1)
        return ()

    lax.fori_loop(0, nchunks, chunk_body, (), unroll=False)
    iv = ir_ref[...]
    zero = jnp.zeros((rb, 1), jnp.float32)
    idx_ref[...] = jnp.concatenate(
        [iv[:, :_ATOM_K], zero], axis=1).astype(jnp.int32)


def _knn(pos, batch32, rb=512, cb=1024):
    n = pos.shape[0]
    pos_pad = jnp.pad(pos, ((0, 0), (0, 5)))
    posT = jnp.pad(pos.T, ((0, 5), (0, 0)))
    bc = batch32[:, None]
    br = batch32[None, :]
    nchunks = n // cb
    grid = (n // rb,)
    out = pl.pallas_call(
        functools.partial(_knn_idx, rb=rb, n=n, cb=cb),
        grid=grid,
        in_specs=[
            pl.BlockSpec((rb, 8), lambda i: (i, 0)),
            pl.BlockSpec((8, n), lambda i: (0, 0)),
            pl.BlockSpec((rb, 1), lambda i: (i, 0)),
            pl.BlockSpec((1, n), lambda i: (0, 0)),
            pl.BlockSpec(memory_space=pltpu.SMEM),
        ],
        out_specs=pl.BlockSpec((rb, 16), lambda i: (i, 0)),
        out_shape=jax.ShapeDtypeStruct((n, 16), jnp.int32),
        scratch_shapes=[
            pltpu.VMEM((rb, 16), jnp.float32),
            pltpu.VMEM((rb, 16), jnp.float32),
            pltpu.VMEM((rb, cb), jnp.float32),
        ],
    )(pos_pad, posT, bc, br, batch32)
    return out[:, :_ATOM_K]


def _sc_gather(table, idx_flat):
    b = idx_flat.shape[0]
    d = table.shape[1]
    info = plsc.get_sparse_core_info()
    nw = info.num_cores * info.num_subcores
    nc = info.num_cores
    b_per_w = b // nw
    chunk = 640
    nchunks = b_per_w // chunk
    mesh = plsc.VectorSubcoreMesh(core_axis_name="c", subcore_axis_name="s")

    @functools.partial(
        pl.kernel, mesh=mesh,
        out_type=jax.ShapeDtypeStruct((b, d), jnp.float32),
        scratch_types=[
            pltpu.VMEM((chunk,), jnp.int32),
            pltpu.VMEM((chunk, d), jnp.float32),
            pltpu.SemaphoreType.DMA,
        ],
    )
    def gather_k(table_hbm, idx_hbm, out_hbm, idx_v, rows_v, sem):
        wid = lax.axis_index("s") * nc + lax.axis_index("c")
        base = wid * b_per_w
        for c in range(nchunks):
            off = base + c * chunk
            pltpu.sync_copy(idx_hbm.at[pl.ds(off, chunk)], idx_v)
            pltpu.async_copy(table_hbm.at[idx_v], rows_v, sem).wait()
            pltpu.sync_copy(rows_v, out_hbm.at[pl.ds(off, chunk)])

    return gather_k(table, idx_flat)


def _attn_body(x_r, pos_r, xg_r, cen, W1e, b1e, W2e, b2e, Wq, Wk, We, Wv, Wo,
               out_ref, *, rb):
    f32 = jnp.float32
    q = jnp.dot(x_r[...], Wq[...], preferred_element_type=f32)
    pr = pos_r[...]
    xga = xg_r[...]
    acc = None
    for dd in range(3):
        diff = xga[:, :, 64 + dd:65 + dd] - pr[None, :, dd:dd + 1]
        sq = diff * diff
        acc = sq if acc is None else acc + sq
    dist = jnp.sqrt(acc + 1e-8)
    delta = dist - cen[...][None, :, :]
    rbf = jnp.exp(-2.0 * delta * delta).reshape(_ATOM_K * rb, _EDGE_DIM)
    h1 = jnp.maximum(
        jnp.dot(rbf, W1e[...], preferred_element_type=f32) + b1e[...], 0.0)
    h = jnp.dot(h1, W2e[...], preferred_element_type=f32) + b2e[...]
    xn_all = xga[:, :, :64].reshape(_ATOM_K * rb, 64)
    km_all = (jnp.dot(xn_all, Wk[...], preferred_element_type=f32)
              + jnp.dot(h, We[...], preferred_element_type=f32))
    v_all = jnp.dot(xn_all, Wv[...], preferred_element_type=f32)
    km3 = km_all.reshape(_ATOM_K, rb, _HID)
    v3 = v_all.reshape(_ATOM_K, rb, _HID)
    svec = [jnp.sum(q * km3[j], axis=1, keepdims=True) / math.sqrt(float(_HID))
            for j in range(_ATOM_K)]
    sc_all = jnp.concatenate(svec, axis=1)
    m = jnp.max(sc_all, axis=1, keepdims=True)
    e = jnp.exp(sc_all - m)
    alpha = e / jnp.sum(e, axis=1, keepdims=True)
    agg = None
    for j in range(_ATOM_K):
        t = alpha[:, j:j + 1] * v3[j]
        agg = t if agg is None else agg + t
    out_ref[...] = jnp.dot(agg, Wo[...], preferred_element_type=f32)


def _attn(x, pos, xg3, cen, W1e, b1e, W2e, b2e, Wq, Wk, We, Wv, Wo, rb=256):
    n, feat = x.shape
    hid = Wq.shape[1]
    out_dim = Wo.shape[1]
    pos_pad = jnp.pad(pos, ((0, 0), (0, 5)))
    grid = (n // rb,)
    full = lambda shape: pl.BlockSpec(shape, lambda i: tuple(0 for _ in shape))
    return pl.pallas_call(
        functools.partial(_attn_body, rb=rb),
        grid=grid,
        in_specs=[
            pl.BlockSpec((rb, feat), lambda i: (i, 0)),
            pl.BlockSpec((rb, 8), lambda i: (i, 0)),
            pl.BlockSpec((_ATOM_K, rb, 128), lambda i: (0, i, 0)),
            full((1, 8)),
            full((8, 8)), full((1, 8)), full((8, 8)), full((1, 8)),
            full((feat, hid)), full((feat, hid)), full((8, hid)),
            full((feat, hid)), full((hid, out_dim)),
        ],
        out_specs=pl.BlockSpec((rb, out_dim), lambda i: (i, 0)),
        out_shape=jax.ShapeDtypeStruct((n, out_dim), jnp.float32),
    )(x, pos_pad, xg3, cen, W1e, b1e, W2e, b2e, Wq, Wk, We, Wv, Wo)


def kernel(x, pos, batch, W1e, b1e, W2e, b2e, Wq, Wk, We, Wv, Wo):
    n, feat = x.shape
    batch32 = batch.astype(jnp.int32)
    idx = _knn(pos, batch32)
    idx_flat = idx.T.reshape(-1).astype(jnp.int32)
    table = jnp.concatenate(
        [x, pos, jnp.zeros((n, 128 - feat - 3), jnp.float32)], axis=1)
    xg = _sc_gather(table, idx_flat)
    xg3 = xg.reshape(_ATOM_K, n, 128)
    cen = jnp.linspace(0.0, 6.0, _EDGE_DIM, dtype=jnp.float32)[None, :]
    return _attn(x, pos, xg3, cen, W1e, b1e[None, :], W2e, b2e[None, :],
                 Wq, Wk, We, Wv, Wo)


# speedup vs baseline: 1.2501x; 1.0074x over previous
import functools
import math

import jax
import jax.numpy as jnp
from jax import lax
from jax.experimental import pallas as pl
from jax.experimental.pallas import tpu as pltpu
from jax.experimental.pallas import tpu_sc as plsc

_ATOM_K = 15
_EDGE_DIM = 8
_HID = 128


_BIG = 2 ** 30


def _knn_idx(pos_r, posT, bc, br, bs, idx_ref, dr_ref, ir_ref, scr, *,
             rb, n, cb):
    i = pl.program_id(0)
    nchunks = n // cb
    inf = jnp.inf
    fake = 1e30
    pr = pos_r[...]
    bcv = bc[...]
    rid = jax.lax.broadcasted_iota(jnp.int32, (rb, 1), 0) + i * rb
    dr_ref[...] = jnp.full((rb, 16), inf, jnp.float32)
    ir_ref[...] = jnp.full((rb, 16), fake, jnp.float32)
    bmin_r = bs[i * rb]
    bmax_r = bs[i * rb + rb - 1]
    bvals = jax.lax.broadcasted_iota(jnp.int32, (16, 1), 0)
    cnt = jnp.sum((bvals == br[...]).astype(jnp.float32), axis=1,
                  keepdims=True)
    need0 = jnp.max(((bvals >= bmin_r) & (bvals <= bmax_r)
                     & (cnt < 16.0)).astype(jnp.int32)) > 0

    def chunk_body(c, _):
        lo = pl.multiple_of(c * cb, cb)
        overlap = (bs[lo] <= bmax_r) & (bs[lo + cb - 1] >= bmin_r)

        @pl.when(overlap | ((c == 0) & need0))
        def _():
            acc = None
            for d in range(3):
                diff = pr[:, d:d + 1] - posT[d:d + 1, pl.ds(lo, cb)]
                sq = diff * diff
                acc = sq if acc is None else acc + sq
            cidc = jax.lax.broadcasted_iota(jnp.int32, (1, cb), 1) + lo
            cidf = cidc.astype(jnp.float32)
            valid = (bcv == br[:, pl.ds(lo, cb)]) & (rid != cidc)
            scr[...] = jnp.where(valid, acc, inf)
            cd_cols, ci_cols = [], []
            for j in range(_ATOM_K):
                v = scr[...]
                m = jnp.min(v, axis=1, keepdims=True)
                cand = jnp.where(v == m, cidf, fake)
                am = jnp.min(cand, axis=1, keepdims=True)
                cd_cols.append(m)
                ci_cols.append(am)
                if j + 1 < _ATOM_K:
                    scr[...] = jnp.where(cidf == am, inf, v)
            pad_d = jnp.full((rb, 1), inf, jnp.float32)
            pad_i = jnp.full((rb, 1), fake, jnp.float32)
            d_cat = jnp.concatenate([dr_ref[...]] + cd_cols + [pad_d], axis=1)
            i_cat = jnp.concatenate([ir_ref[...]] + ci_cols + [pad_i], axis=1)
            nd, ni = [], []
            for j in range(_ATOM_K):
                m2 = jnp.min(d_cat, axis=1, keepdims=True)
                c2 = jnp.where(d_cat == m2, i_cat, fake)
                am2 = jnp.min(c2, axis=1, keepdims=True)
                nd.append(m2)
                ni.append(am2)
                if j + 1 < _ATOM_K:
                    d_cat = jnp.where((d_cat == m2) & (i_cat == am2), inf,
                                      d_cat)
            dr_ref[...] = jnp.concatenate(nd + [pad_d], axis=1)
            ir_ref[...] = jnp.concatenate(ni + [pad_i], axis=1)
        return ()

    lax.fori_loop(0, nchunks, chunk_body, (), unroll=False)
    iv = ir_ref[...]
    zero = jnp.zeros((rb, 1), jnp.float32)
    idx_ref[...] = jnp.concatenate(
        [iv[:, :_ATOM_K], zero], axis=1).astype(jnp.int32)


def _knn(pos, batch32, rb=512, cb=512):
    n = pos.shape[0]
    pos_pad = jnp.pad(pos, ((0, 0), (0, 5)))
    posT = jnp.pad(pos.T, ((0, 5), (0, 0)))
    bc = batch32[:, None]
    br = batch32[None, :]
    nchunks = n // cb
    grid = (n // rb,)
    out = pl.pallas_call(
        functools.partial(_knn_idx, rb=rb, n=n, cb=cb),
        grid=grid,
        in_specs=[
            pl.BlockSpec((rb, 8), lambda i: (i, 0)),
            pl.BlockSpec((8, n), lambda i: (0, 0)),
            pl.BlockSpec((rb, 1), lambda i: (i, 0)),
            pl.BlockSpec((1, n), lambda i: (0, 0)),
            pl.BlockSpec(memory_space=pltpu.SMEM),
        ],
        out_specs=pl.BlockSpec((rb, 16), lambda i: (i, 0)),
        out_shape=jax.ShapeDtypeStruct((n, 16), jnp.int32),
        scratch_shapes=[
            pltpu.VMEM((rb, 16), jnp.float32),
            pltpu.VMEM((rb, 16), jnp.float32),
            pltpu.VMEM((rb, cb), jnp.float32),
        ],
    )(pos_pad, posT, bc, br, batch32)
    return out[:, :_ATOM_K]


def _sc_gather(table, idx_flat):
    b = idx_flat.shape[0]
    d = table.shape[1]
    info = plsc.get_sparse_core_info()
    nw = info.num_cores * info.num_subcores
    nc = info.num_cores
    b_per_w = b // nw
    chunk = 640
    nchunks = b_per_w // chunk
    mesh = plsc.VectorSubcoreMesh(core_axis_name="c", subcore_axis_name="s")

    @functools.partial(
        pl.kernel, mesh=mesh,
        out_type=jax.ShapeDtypeStruct((b, d), jnp.float32),
        scratch_types=[
            pltpu.VMEM((chunk,), jnp.int32),
            pltpu.VMEM((chunk, d), jnp.float32),
            pltpu.SemaphoreType.DMA,
        ],
    )
    def gather_k(table_hbm, idx_hbm, out_hbm, idx_v, rows_v, sem):
        wid = lax.axis_index("s") * nc + lax.axis_index("c")
        base = wid * b_per_w
        for c in range(nchunks):
            off = base + c * chunk
            pltpu.sync_copy(idx_hbm.at[pl.ds(off, chunk)], idx_v)
            pltpu.async_copy(table_hbm.at[idx_v], rows_v, sem).wait()
            pltpu.sync_copy(rows_v, out_hbm.at[pl.ds(off, chunk)])

    return gather_k(table, idx_flat)


def _attn_body(x_r, pos_r, xg_r, cen, W1e, b1e, W2e, b2e, Wq, Wk, We, Wv, Wo,
               out_ref, *, rb):
    f32 = jnp.float32
    q = jnp.dot(x_r[...], Wq[...], preferred_element_type=f32)
    pr = pos_r[...]
    xga = xg_r[...]
    acc = None
    for dd in range(3):
        diff = xga[:, :, 64 + dd:65 + dd] - pr[None, :, dd:dd + 1]
        sq = diff * diff
        acc = sq if acc is None else acc + sq
    dist = jnp.sqrt(acc + 1e-8)
    delta = dist - cen[...][None, :, :]
    rbf = jnp.exp(-2.0 * delta * delta).reshape(_ATOM_K * rb, _EDGE_DIM)
    h1 = jnp.maximum(
        jnp.dot(rbf, W1e[...], preferred_element_type=f32) + b1e[...], 0.0)
    h = jnp.dot(h1, W2e[...], preferred_element_type=f32) + b2e[...]
    xn_all = xga[:, :, :64].reshape(_ATOM_K * rb, 64)
    km_all = (jnp.dot(xn_all, Wk[...], preferred_element_type=f32)
              + jnp.dot(h, We[...], preferred_element_type=f32))
    v_all = jnp.dot(xn_all, Wv[...], preferred_element_type=f32)
    km3 = km_all.reshape(_ATOM_K, rb, _HID)
    v3 = v_all.reshape(_ATOM_K, rb, _HID)
    svec = [jnp.sum(q * km3[j], axis=1, keepdims=True) / math.sqrt(float(_HID))
            for j in range(_ATOM_K)]
    sc_all = jnp.concatenate(svec, axis=1)
    m = jnp.max(sc_all, axis=1, keepdims=True)
    e = jnp.exp(sc_all - m)
    alpha = e / jnp.sum(e, axis=1, keepdims=True)
    agg = None
    for j in range(_ATOM_K):
        t = alpha[:, j:j + 1] * v3[j]
        agg = t if agg is None else agg + t
    out_ref[...] = jnp.dot(agg, Wo[...], preferred_element_type=f32)


def _attn(x, pos, xg3, cen, W1e, b1e, W2e, b2e, Wq, Wk, We, Wv, Wo, rb=256):
    n, feat = x.shape
    hid = Wq.shape[1]
    out_dim = Wo.shape[1]
    pos_pad = jnp.pad(pos, ((0, 0), (0, 5)))
    grid = (n // rb,)
    full = lambda shape: pl.BlockSpec(shape, lambda i: tuple(0 for _ in shape))
    return pl.pallas_call(
        functools.partial(_attn_body, rb=rb),
        grid=grid,
        in_specs=[
            pl.BlockSpec((rb, feat), lambda i: (i, 0)),
            pl.BlockSpec((rb, 8), lambda i: (i, 0)),
            pl.BlockSpec((_ATOM_K, rb, 128), lambda i: (0, i, 0)),
            full((1, 8)),
            full((8, 8)), full((1, 8)), full((8, 8)), full((1, 8)),
            full((feat, hid)), full((feat, hid)), full((8, hid)),
            full((feat, hid)), full((hid, out_dim)),
        ],
        out_specs=pl.BlockSpec((rb, out_dim), lambda i: (i, 0)),
        out_shape=jax.ShapeDtypeStruct((n, out_dim), jnp.float32),
    )(x, pos_pad, xg3, cen, W1e, b1e, W2e, b2e, Wq, Wk, We, Wv, Wo)


def kernel(x, pos, batch, W1e, b1e, W2e, b2e, Wq, Wk, We, Wv, Wo):
    n, feat = x.shape
    batch32 = batch.astype(jnp.int32)
    idx = _knn(pos, batch32)
    idx_flat = idx.T.reshape(-1).astype(jnp.int32)
    table = jnp.concatenate(
        [x, pos, jnp.zeros((n, 128 - feat - 3), jnp.float32)], axis=1)
    xg = _sc_gather(table, idx_flat)
    xg3 = xg.reshape(_ATOM_K, n, 128)
    cen = jnp.linspace(0.0, 6.0, _EDGE_DIM, dtype=jnp.float32)[None, :]
    return _attn(x, pos, xg3, cen, W1e, b1e[None, :], W2e, b2e[None, :],
                 Wq, Wk, We, Wv, Wo)
